# baseline (speedup 1.0000x reference)
import functools

import jax
import jax.numpy as jnp
from jax import lax
from jax.experimental import pallas as pl
from jax.experimental.pallas import tpu as pltpu
from jax.experimental.pallas import tpu_sc as plsc

_N = 10000
_D = 128
_NC = 2
_NS = 16
_NW = _NC * _NS
_NPAD = 10112
_RPT = _NPAD // _NS
_CHUNK = 128
_CHUNKS_PER_W = 80
_EPW = _CHUNK * _CHUNKS_PER_W
_EPAD = _EPW * _NW
_CNTW = 16
_WCHUNKS = [_CHUNK] * (_RPT // _CHUNK) + ([_RPT % _CHUNK] if _RPT % _CHUNK else [])
_NBUF = 2
_RBLK = 1000


def _matmul_body(x_ref, w_ref, y_ref):
    y_ref[...] = jnp.dot(x_ref[...], w_ref[...], preferred_element_type=jnp.float32)


def _matmul(x, w):
    return pl.pallas_call(
        _matmul_body,
        grid=(_N // _RBLK,),
        in_specs=[
            pl.BlockSpec((_RBLK, _D), lambda i: (i, 0)),
            pl.BlockSpec((_D, _D), lambda i: (0, 0)),
        ],
        out_specs=pl.BlockSpec((_RBLK, _D), lambda i: (i, 0)),
        out_shape=jax.ShapeDtypeStruct((_N, _D), jnp.float32),
    )(x, w)


def _combine_body(a0_ref, a1_ref, c0_ref, c1_ref, x_ref, w_ref, b_ref, o_ref):
    cnt = jnp.maximum(c0_ref[:, 0:1] + c1_ref[:, 0:1], 1.0)
    mean = (a0_ref[...] + a1_ref[...]) / cnt
    z = mean + jnp.dot(x_ref[...], w_ref[...], preferred_element_type=jnp.float32) + b_ref[...]
    o_ref[...] = jnp.maximum(z, 0.0)


def _combine(a0, a1, c0, c1, x, w, b):
    return pl.pallas_call(
        _combine_body,
        grid=(_N // _RBLK,),
        in_specs=[
            pl.BlockSpec((_RBLK, _D), lambda i: (i, 0)),
            pl.BlockSpec((_RBLK, _D), lambda i: (i, 0)),
            pl.BlockSpec((_RBLK, _D), lambda i: (i, 0)),
            pl.BlockSpec((_RBLK, _D), lambda i: (i, 0)),
            pl.BlockSpec((_RBLK, _D), lambda i: (i, 0)),
            pl.BlockSpec((_D, _D), lambda i: (0, 0)),
            pl.BlockSpec((1, _D), lambda i: (0, 0)),
        ],
        out_specs=pl.BlockSpec((_RBLK, _D), lambda i: (i, 0)),
        out_shape=jax.ShapeDtypeStruct((_N, _D), jnp.float32),
    )(a0, a1, c0, c1, x, w, b)


def _sc_acc_body(y_hbm, ei_hbm, zrow_hbm, acc_hbm, *scr):
    idx = list(scr[0:_NBUF])
    rows = list(scr[_NBUF:2 * _NBUF])
    acc_sh = scr[2 * _NBUF]
    sem = list(scr[2 * _NBUF + 1:])
    c = lax.axis_index("c")
    s = lax.axis_index("s")
    wid = c * _NS + s
    rbase = s * _RPT
    cbase = wid * _CHUNKS_PER_W
    rows_a = rows[0]

    pltpu.sync_copy(zrow_hbm, rows_a)
    for k, n in enumerate(_WCHUNKS):
        pltpu.sync_copy(rows_a.at[pl.ds(0, n)],
                        acc_sh.at[pl.ds(rbase + k * _CHUNK, n)])
    plsc.subcore_barrier()

    for t in range(_NBUF):
        pltpu.sync_copy(ei_hbm.at[cbase + t], idx[t])
        pltpu.async_copy(y_hbm.at[idx[t].at[0]], rows[t], sem[t])

    def body(i, carry):
        for t in range(_NBUF):
            j = _NBUF * i + t
            pltpu.make_async_copy(y_hbm.at[idx[t].at[0]], rows[t],
                                  sem[t]).wait()
            pltpu.sync_copy(rows[t], acc_sh.at[idx[t].at[1]], add=True)
            jn = jnp.minimum(j + _NBUF, _CHUNKS_PER_W - 1)
            pltpu.sync_copy(ei_hbm.at[cbase + jn], idx[t])
            pltpu.async_copy(y_hbm.at[idx[t].at[0]], rows[t], sem[t])
        return carry

    lax.fori_loop(0, _CHUNKS_PER_W // _NBUF, body, 0)
    for t in range(_NBUF):
        pltpu.make_async_copy(y_hbm.at[idx[t].at[0]], rows[t], sem[t]).wait()
    plsc.subcore_barrier()

    obase = c * _NPAD + rbase
    for k, n in enumerate(_WCHUNKS):
        pltpu.sync_copy(acc_sh.at[pl.ds(rbase + k * _CHUNK, n)],
                        rows_a.at[pl.ds(0, n)])
        pltpu.sync_copy(rows_a.at[pl.ds(0, n)],
                        acc_hbm.at[pl.ds(obase + k * _CHUNK, n)])


def _sc_cnt_body(dst_hbm, ones_hbm, zrow_hbm,
                 cnt_hbm,
                 dst_all, ones_v, cnt_sh):
    c = lax.axis_index("c")
    s = lax.axis_index("s")
    wid = c * _NS + s
    rbase = s * _RPT

    pltpu.sync_copy(dst_hbm.at[wid], dst_all)
    pltpu.sync_copy(zrow_hbm, ones_v)
    for k, n in enumerate(_WCHUNKS):
        pltpu.sync_copy(ones_v.at[pl.ds(0, n)],
                        cnt_sh.at[pl.ds(rbase + k * _CHUNK, n)])
    pltpu.sync_copy(ones_hbm, ones_v)
    plsc.subcore_barrier()

    def body(j, carry):
        pltpu.sync_copy(ones_v, cnt_sh.at[dst_all.at[j]], add=True)
        return carry

    lax.fori_loop(0, _CHUNKS_PER_W, body, 0)
    plsc.subcore_barrier()

    obase = c * _NPAD + rbase
    for k, n in enumerate(_WCHUNKS):
        pltpu.sync_copy(cnt_sh.at[pl.ds(rbase + k * _CHUNK, n)],
                        ones_v.at[pl.ds(0, n)])
        pltpu.sync_copy(ones_v.at[pl.ds(0, n)],
                        cnt_hbm.at[pl.ds(obase + k * _CHUNK, n)])


@functools.cache
def _sc_acc_call():
    return functools.partial(
        pl.kernel,
        mesh=plsc.VectorSubcoreMesh(core_axis_name="c", subcore_axis_name="s"),
        out_type=jax.ShapeDtypeStruct((_NC * _NPAD, _D), jnp.float32),
        scratch_types=(
            [pltpu.VMEM((2, _CHUNK), jnp.int32)] * _NBUF
            + [pltpu.VMEM((_CHUNK, _D), jnp.float32)] * _NBUF
            + [pltpu.VMEM_SHARED((_NPAD, _D), jnp.float32)]
            + [pltpu.SemaphoreType.DMA] * _NBUF
        ),
    )(_sc_acc_body)


@functools.cache
def _sc_cnt_call(w=_D):
    return functools.partial(
        pl.kernel,
        mesh=plsc.VectorSubcoreMesh(core_axis_name="c", subcore_axis_name="s"),
        out_type=jax.ShapeDtypeStruct((_NC * _NPAD, w), jnp.float32),
        scratch_types=[
            pltpu.VMEM((_CHUNKS_PER_W, _CHUNK), jnp.int32),
            pltpu.VMEM((_CHUNK, w), jnp.float32),
            pltpu.VMEM_SHARED((_NPAD, w), jnp.float32),
        ],
    )(_sc_cnt_body)


def kernel(x, edge_index, W_l, W_r, b_l):
    src = edge_index[0].astype(jnp.int32)
    dst = edge_index[1].astype(jnp.int32)
    e = src.shape[0]
    pad = _EPAD - e
    src = jnp.concatenate([src, jnp.zeros((pad,), jnp.int32)])
    dst = jnp.concatenate([dst, jnp.full((pad,), _NPAD - 1, jnp.int32)])
    ei = jnp.stack([src.reshape(-1, _CHUNK), dst.reshape(-1, _CHUNK)], axis=1)
    dst3 = dst.reshape(_NW, _CHUNKS_PER_W, _CHUNK)

    y = _matmul(x, W_l)

    ones = jnp.ones((_CHUNK, _D), jnp.float32)
    zrow = jnp.zeros((_CHUNK, _D), jnp.float32)
    acc = _sc_acc_call()(y, ei, zrow)
    cnt = _sc_cnt_call(_D)(dst3, ones, zrow)

    a0 = acc[:_N]
    a1 = acc[_NPAD:_NPAD + _N]
    c0 = cnt[:_N]
    c1 = cnt[_NPAD:_NPAD + _N]
    return _combine(a0, a1, c0, c1, x, W_r, b_l.reshape(1, _D))


# speedup vs baseline: 1.0001x; 1.0001x over previous
import functools

import jax
import jax.numpy as jnp
from jax import lax
from jax.experimental import pallas as pl
from jax.experimental.pallas import tpu as pltpu
from jax.experimental.pallas import tpu_sc as plsc

_N = 10000
_D = 128
_NC = 2
_NS = 16
_NW = _NC * _NS
_NPAD = 10112
_RPT = _NPAD // _NS
_CHUNK = 128
_CHUNKS_PER_W = 80
_EPW = _CHUNK * _CHUNKS_PER_W
_EPAD = _EPW * _NW
_CNTW = 16
_WCHUNKS = [_CHUNK] * (_RPT // _CHUNK) + ([_RPT % _CHUNK] if _RPT % _CHUNK else [])
_NBUF = 2
_NIDX = 4
_RBLK = 1000


def _matmul_body(x_ref, w_ref, y_ref):
    y_ref[...] = jnp.dot(x_ref[...], w_ref[...], preferred_element_type=jnp.float32)


def _matmul(x, w):
    return pl.pallas_call(
        _matmul_body,
        grid=(_N // _RBLK,),
        in_specs=[
            pl.BlockSpec((_RBLK, _D), lambda i: (i, 0)),
            pl.BlockSpec((_D, _D), lambda i: (0, 0)),
        ],
        out_specs=pl.BlockSpec((_RBLK, _D), lambda i: (i, 0)),
        out_shape=jax.ShapeDtypeStruct((_N, _D), jnp.float32),
    )(x, w)


def _combine_body(a0_ref, a1_ref, c0_ref, c1_ref, x_ref, w_ref, b_ref, o_ref):
    cnt = jnp.maximum(c0_ref[:, 0:1] + c1_ref[:, 0:1], 1.0)
    mean = (a0_ref[...] + a1_ref[...]) / cnt
    z = mean + jnp.dot(x_ref[...], w_ref[...], preferred_element_type=jnp.float32) + b_ref[...]
    o_ref[...] = jnp.maximum(z, 0.0)


def _combine(a0, a1, c0, c1, x, w, b):
    return pl.pallas_call(
        _combine_body,
        grid=(_N // _RBLK,),
        in_specs=[
            pl.BlockSpec((_RBLK, _D), lambda i: (i, 0)),
            pl.BlockSpec((_RBLK, _D), lambda i: (i, 0)),
            pl.BlockSpec((_RBLK, _D), lambda i: (i, 0)),
            pl.BlockSpec((_RBLK, _D), lambda i: (i, 0)),
            pl.BlockSpec((_RBLK, _D), lambda i: (i, 0)),
            pl.BlockSpec((_D, _D), lambda i: (0, 0)),
            pl.BlockSpec((1, _D), lambda i: (0, 0)),
        ],
        out_specs=pl.BlockSpec((_RBLK, _D), lambda i: (i, 0)),
        out_shape=jax.ShapeDtypeStruct((_N, _D), jnp.float32),
    )(a0, a1, c0, c1, x, w, b)


def _sc_acc_body(y_hbm, ei_hbm, zrow_hbm, acc_hbm, *scr):
    idx = list(scr[0:_NIDX])
    rows = list(scr[_NIDX:_NIDX + _NBUF])
    acc_sh = scr[_NIDX + _NBUF]
    sem = list(scr[_NIDX + _NBUF + 1:_NIDX + 2 * _NBUF + 1])
    isem = list(scr[_NIDX + 2 * _NBUF + 1:])
    c = lax.axis_index("c")
    s = lax.axis_index("s")
    wid = c * _NS + s
    rbase = s * _RPT
    cbase = wid * _CHUNKS_PER_W
    rows_a = rows[0]

    pltpu.sync_copy(zrow_hbm, rows_a)
    for k, n in enumerate(_WCHUNKS):
        pltpu.sync_copy(rows_a.at[pl.ds(0, n)],
                        acc_sh.at[pl.ds(rbase + k * _CHUNK, n)])
    plsc.subcore_barrier()

    pltpu.sync_copy(ei_hbm.at[cbase + 0], idx[0])
    pltpu.sync_copy(ei_hbm.at[cbase + 1], idx[1])
    pltpu.async_copy(ei_hbm.at[cbase + 2], idx[2], isem[2])
    pltpu.async_copy(ei_hbm.at[cbase + 3], idx[3], isem[3])
    pltpu.async_copy(y_hbm.at[idx[0].at[0]], rows[0], sem[0])
    pltpu.async_copy(y_hbm.at[idx[1].at[0]], rows[1], sem[1])

    def body(i, carry):
        for t in range(_NIDX):
            j = _NIDX * i + t
            r = t % _NBUF
            q2 = (t + 2) % _NIDX
            pltpu.make_async_copy(y_hbm.at[idx[t].at[0]], rows[r],
                                  sem[r]).wait()
            pltpu.sync_copy(rows[r], acc_sh.at[idx[t].at[1]], add=True)
            jn = jnp.minimum(j + _NIDX, _CHUNKS_PER_W - 1)
            pltpu.async_copy(ei_hbm.at[cbase + jn], idx[t], isem[t])
            pltpu.make_async_copy(ei_hbm.at[cbase], idx[q2], isem[q2]).wait()
            pltpu.async_copy(y_hbm.at[idx[q2].at[0]], rows[r], sem[r])
        return carry

    lax.fori_loop(0, _CHUNKS_PER_W // _NIDX, body, 0)
    for r in range(_NBUF):
        pltpu.make_async_copy(y_hbm.at[idx[0].at[0]], rows[r], sem[r]).wait()
    for q in (2, 3):
        pltpu.make_async_copy(ei_hbm.at[cbase], idx[q], isem[q]).wait()
    plsc.subcore_barrier()

    obase = c * _NPAD + rbase
    for k, n in enumerate(_WCHUNKS):
        pltpu.sync_copy(acc_sh.at[pl.ds(rbase + k * _CHUNK, n)],
                        rows_a.at[pl.ds(0, n)])
        pltpu.sync_copy(rows_a.at[pl.ds(0, n)],
                        acc_hbm.at[pl.ds(obase + k * _CHUNK, n)])


def _sc_cnt_body(dst_hbm, ones_hbm, zrow_hbm,
                 cnt_hbm,
                 dst_all, ones_v, cnt_sh):
    c = lax.axis_index("c")
    s = lax.axis_index("s")
    wid = c * _NS + s
    rbase = s * _RPT

    pltpu.sync_copy(dst_hbm.at[wid], dst_all)
    pltpu.sync_copy(zrow_hbm, ones_v)
    for k, n in enumerate(_WCHUNKS):
        pltpu.sync_copy(ones_v.at[pl.ds(0, n)],
                        cnt_sh.at[pl.ds(rbase + k * _CHUNK, n)])
    pltpu.sync_copy(ones_hbm, ones_v)
    plsc.subcore_barrier()

    def body(j, carry):
        pltpu.sync_copy(ones_v, cnt_sh.at[dst_all.at[j]], add=True)
        return carry

    lax.fori_loop(0, _CHUNKS_PER_W, body, 0)
    plsc.subcore_barrier()

    obase = c * _NPAD + rbase
    for k, n in enumerate(_WCHUNKS):
        pltpu.sync_copy(cnt_sh.at[pl.ds(rbase + k * _CHUNK, n)],
                        ones_v.at[pl.ds(0, n)])
        pltpu.sync_copy(ones_v.at[pl.ds(0, n)],
                        cnt_hbm.at[pl.ds(obase + k * _CHUNK, n)])


@functools.cache
def _sc_acc_call():
    return functools.partial(
        pl.kernel,
        mesh=plsc.VectorSubcoreMesh(core_axis_name="c", subcore_axis_name="s"),
        out_type=jax.ShapeDtypeStruct((_NC * _NPAD, _D), jnp.float32),
        scratch_types=(
            [pltpu.VMEM((2, _CHUNK), jnp.int32)] * _NIDX
            + [pltpu.VMEM((_CHUNK, _D), jnp.float32)] * _NBUF
            + [pltpu.VMEM_SHARED((_NPAD, _D), jnp.float32)]
            + [pltpu.SemaphoreType.DMA] * _NBUF
            + [pltpu.SemaphoreType.DMA] * _NIDX
        ),
    )(_sc_acc_body)


@functools.cache
def _sc_cnt_call(w=_D):
    return functools.partial(
        pl.kernel,
        mesh=plsc.VectorSubcoreMesh(core_axis_name="c", subcore_axis_name="s"),
        out_type=jax.ShapeDtypeStruct((_NC * _NPAD, w), jnp.float32),
        scratch_types=[
            pltpu.VMEM((_CHUNKS_PER_W, _CHUNK), jnp.int32),
            pltpu.VMEM((_CHUNK, w), jnp.float32),
            pltpu.VMEM_SHARED((_NPAD, w), jnp.float32),
        ],
    )(_sc_cnt_body)


def kernel(x, edge_index, W_l, W_r, b_l):
    src = edge_index[0].astype(jnp.int32)
    dst = edge_index[1].astype(jnp.int32)
    e = src.shape[0]
    pad = _EPAD - e
    src = jnp.concatenate([src, jnp.zeros((pad,), jnp.int32)])
    dst = jnp.concatenate([dst, jnp.full((pad,), _NPAD - 1, jnp.int32)])
    ei = jnp.stack([src.reshape(-1, _CHUNK), dst.reshape(-1, _CHUNK)], axis=1)
    dst3 = dst.reshape(_NW, _CHUNKS_PER_W, _CHUNK)

    y = _matmul(x, W_l)

    ones = jnp.ones((_CHUNK, _D), jnp.float32)
    zrow = jnp.zeros((_CHUNK, _D), jnp.float32)
    cnt = _sc_cnt_call(_D)(dst3, ones, zrow)
    acc = _sc_acc_call()(y, ei, zrow)

    a0 = acc[:_N]
    a1 = acc[_NPAD:_NPAD + _N]
    c0 = cnt[:_N]
    c1 = cnt[_NPAD:_NPAD + _N]
    return _combine(a0, a1, c0, c1, x, W_r, b_l.reshape(1, _D))
